# baseline (speedup 1.0000x reference)
import functools

import jax
import jax.numpy as jnp
from jax import lax
from jax.experimental import pallas as pl
from jax.experimental.pallas import tpu as pltpu
from jax.experimental.pallas import tpu_sc as plsc

_N_USERS = 25000
_N_ITEMS = 25000
_N = _N_USERS + _N_ITEMS
_E = 800000
_D = 64
_DF = 128
_NLAYERS = 3
_BATCH = 1024

_NC = 2
_NS = 16
_EPAD = 819200
_ER = _EPAD // 128
_RPT = _ER // _NS
_CHUNKS = _RPT // 8
_NPAD = 50176
_RNG = _NPAD // 4
_WPT = _RNG // _NS

_mesh = plsc.VectorSubcoreMesh(core_axis_name="c", subcore_axis_name="s")


@functools.partial(
    pl.kernel,
    out_type=jax.ShapeDtypeStruct((_NPAD, _DF), jnp.float32),
    mesh=_mesh,
    scratch_types=[
        pltpu.VMEM_SHARED((_RNG + 8, _DF), jnp.float32),
        pltpu.VMEM((8, 128), jnp.int32),
        pltpu.VMEM((8, 128), jnp.int32),
        pltpu.VMEM((16, 64), jnp.int32),
        pltpu.VMEM((1024,), jnp.float32),
        pltpu.VMEM((64, _DF), jnp.float32),
        pltpu.VMEM((64, _DF), jnp.float32),
        pltpu.VMEM((64, _DF), jnp.float32),
        pltpu.SemaphoreType.DMA,
        pltpu.SemaphoreType.DMA,
        pltpu.SemaphoreType.DMA,
        pltpu.SemaphoreType.DMA,
        pltpu.SemaphoreType.DMA,
        pltpu.SemaphoreType.DMA,
    ],
)
def _propagate(tin, src2, dst2, val2, tout, acc, sidx, draw, didx, cval,
               rows0, rows1, rows2, g0, g1, g2, s0, s1, s2):
    cid = lax.axis_index("c")
    sid = lax.axis_index("s")
    zero16 = jnp.zeros((16,), jnp.float32)
    rows = (rows0, rows1, rows2)
    gsem = (g0, g1, g2)
    ssem = (s0, s1, s2)

    def zfill(i, carry):
        for q in range(8):
            rows0[i, pl.ds(q * 16, 16)] = zero16
        return carry

    for rl in range(2):
        r = 2 * cid + rl
        lo = r * _RNG
        lax.fori_loop(0, 64, zfill, 0)
        for z in range(12):
            pltpu.sync_copy(rows0, acc.at[pl.ds(sid * _WPT + z * 64, 64)])
        pltpu.sync_copy(rows0.at[pl.ds(0, 16)], acc.at[pl.ds(sid * _WPT + 768, 16)])
        plsc.subcore_barrier()

        def chunk(c, carry):
            base = sid * _RPT + 8 * c
            pltpu.sync_copy(src2.at[pl.ds(base, 8)], sidx)
            pltpu.sync_copy(dst2.at[pl.ds(base, 8)], draw)
            pltpu.sync_copy(val2.at[pl.ds(base * 128, 1024)], cval)

            def locdst(g, ic):
                j = g // 8
                col = (g - j * 8) * 16
                v = draw[j, pl.ds(col, 16)]
                m = (v >= lo) & (v < lo + _RNG)
                didx[g // 4, pl.ds((g - (g // 4) * 4) * 16, 16)] = (
                    jnp.where(m, v - lo, _RNG)
                )
                return ic

            lax.fori_loop(0, 64, locdst, 0)

            def fire_gather(h):
                b = h % 3
                return pltpu.async_copy(
                    tin.at[sidx.at[h // 2, pl.ds((h % 2) * 64, 64)]],
                    rows[b],
                    gsem[b],
                )

            hg = {0: fire_gather(0), 1: fire_gather(1)}
            hs = {}
            for h in range(16):
                b = h % 3
                rb = rows[b]
                hg[h].wait()

                def scale_group(g, ic, _h=h, _rb=rb):
                    val16 = cval[pl.ds(_h * 64 + g * 16, 16)]

                    def lane(l, c2):
                        vs = val16[jnp.full((16,), l, jnp.int32)]
                        e = g * 16 + l
                        for q in range(8):
                            fs = pl.ds(q * 16, 16)
                            _rb[e, fs] = _rb[e, fs] * vs
                        return c2

                    return lax.fori_loop(0, 16, lane, ic)

                lax.fori_loop(0, 4, scale_group, 0)
                hs[h] = pltpu.async_copy(rb, acc.at[didx.at[h]], ssem[b], add=True)
                nx = h + 2
                if nx < 16:
                    if h >= 1:
                        hs[h - 1].wait()
                    hg[nx] = fire_gather(nx)
            hs[13].wait()
            hs[14].wait()
            hs[15].wait()
            return carry

        lax.fori_loop(0, _CHUNKS, chunk, 0)
        plsc.subcore_barrier()
        pltpu.sync_copy(
            acc.at[pl.ds(sid * _WPT, _WPT)],
            tout.at[pl.ds(lo + sid * _WPT, _WPT)],
        )
        plsc.subcore_barrier()


@functools.partial(
    pl.kernel,
    out_type=jax.ShapeDtypeStruct((_NC * _BATCH, _DF), jnp.float32),
    mesh=_mesh,
    scratch_types=[
        pltpu.VMEM((64,), jnp.int32),
        pltpu.VMEM((64, _DF), jnp.float32),
        pltpu.VMEM((64, _DF), jnp.float32),
        pltpu.SemaphoreType.DMA,
    ],
)
def _user_gather(t0, t1, t2, t3, uidx, uout, uidxv, rowbuf, ubuf, usem):
    cid = lax.axis_index("c")
    sid = lax.axis_index("s")
    ub = sid * 64

    pltpu.sync_copy(uidx.at[pl.ds(ub, 64)], uidxv)

    def grab(tab, accumulate):
        pltpu.async_copy(tab.at[uidxv], rowbuf, usem).wait()

        def accrow(rr, carry):
            for q in range(8):
                fs = pl.ds(q * 16, 16)
                if accumulate:
                    ubuf[rr, fs] = ubuf[rr, fs] + rowbuf[rr, fs]
                else:
                    ubuf[rr, fs] = rowbuf[rr, fs]
            return carry

        lax.fori_loop(0, 64, accrow, 0)

    @pl.when(cid == 0)
    def _():
        grab(t0, False)
        grab(t1, True)

    @pl.when(cid == 1)
    def _():
        grab(t2, False)
        grab(t3, True)

    def fin(rr, carry):
        for q in range(8):
            fs = pl.ds(q * 16, 16)
            ubuf[rr, fs] = ubuf[rr, fs] * 0.25
        return carry

    lax.fori_loop(0, 64, fin, 0)
    pltpu.sync_copy(ubuf, uout.at[pl.ds(cid * _BATCH + ub, 64)])


def _items_mean_body(i0_ref, i1_ref, i2_ref, i3_ref, out_ref):
    out_ref[...] = (i0_ref[...] + i1_ref[...] + i2_ref[...] + i3_ref[...]) * 0.25


def _items_mean(t0, t1, t2, t3):
    spec = pl.BlockSpec((1000, _DF), lambda i: (25 + i, 0))
    return pl.pallas_call(
        _items_mean_body,
        grid=(_N_ITEMS // 1000,),
        in_specs=[spec, spec, spec, spec],
        out_specs=pl.BlockSpec((1000, _DF), lambda i: (i, 0)),
        out_shape=jax.ShapeDtypeStruct((_N_ITEMS, _DF), jnp.float32),
    )(t0, t1, t2, t3)


def _rating_body(u_ref, items_ref, rat_ref, orig_ref):
    u = u_ref[0] + u_ref[1]
    ua = u[:, :_D]
    ub = u[:, _D:]
    items_a = items_ref[:, :_D]
    items_b = items_ref[:, _D:]
    dn = (((1,), (1,)), ((), ()))
    a = jax.nn.sigmoid(lax.dot_general(ua, items_a, dn, preferred_element_type=jnp.float32))
    b = jax.nn.sigmoid(lax.dot_general(ub, items_b, dn, preferred_element_type=jnp.float32))
    rat_ref[...] = a + b
    orig_ref[...] = b


_BU = 64


def _rating(u, items_mean):
    return pl.pallas_call(
        _rating_body,
        grid=(_BATCH // _BU,),
        in_specs=[
            pl.BlockSpec((_NC, _BU, _DF), lambda i: (0, i, 0)),
            pl.BlockSpec((_N_ITEMS, _DF), lambda i: (0, 0)),
        ],
        out_specs=[
            pl.BlockSpec((_BU, _N_ITEMS), lambda i: (i, 0)),
            pl.BlockSpec((_BU, _N_ITEMS), lambda i: (i, 0)),
        ],
        out_shape=[
            jax.ShapeDtypeStruct((_BATCH, _N_ITEMS), jnp.float32),
            jax.ShapeDtypeStruct((_BATCH, _N_ITEMS), jnp.float32),
        ],
    )(u, items_mean)


def kernel(users, edge_index, edge_vals, emb_user_conf, emb_user_nonconf,
           emb_item_pop, emb_item_nonpop):
    order = jnp.argsort(edge_index[0])
    src = edge_index[0][order]
    dst = edge_index[1][order]
    edge_vals = edge_vals[order]
    tower_a = jnp.concatenate([emb_user_conf, emb_item_pop], axis=0)
    tower_b = jnp.concatenate([emb_user_nonconf, emb_item_nonpop], axis=0)
    comb = jnp.concatenate([tower_a, tower_b], axis=1)
    t0 = jnp.pad(comb, ((0, _NPAD - _N), (0, 0)))

    pad = _EPAD - _E
    src2 = jnp.pad(src, (0, pad)).reshape(_ER, 128)
    dst2 = jnp.pad(dst, (0, pad)).reshape(_ER, 128)
    val2 = jnp.pad(edge_vals, (0, pad))

    tables = [t0]
    for _ in range(_NLAYERS):
        tables.append(_propagate(tables[-1], src2, dst2, val2))

    u = _user_gather(tables[0], tables[1], tables[2], tables[3], users)
    u = u.reshape(_NC, _BATCH, _DF)

    im = _items_mean(tables[0], tables[1], tables[2], tables[3])
    rating, rating_orig = _rating(u, im)
    return rating, rating_orig


# speedup vs baseline: 1.1415x; 1.1415x over previous
import functools

import jax
import jax.numpy as jnp
from jax import lax
from jax.experimental import pallas as pl
from jax.experimental.pallas import tpu as pltpu
from jax.experimental.pallas import tpu_sc as plsc

_N_USERS = 25000
_N_ITEMS = 25000
_N = _N_USERS + _N_ITEMS
_E = 800000
_D = 64
_DF = 128
_NLAYERS = 3
_BATCH = 1024

_NC = 2
_NS = 16
_EPAD = 819200
_ER = _EPAD // 128
_RPT = _ER // _NS
_CHUNKS = _RPT // 8
_NPAD = 50176
_RNG = _NPAD // 4
_WPT = _RNG // _NS

_mesh = plsc.VectorSubcoreMesh(core_axis_name="c", subcore_axis_name="s")


@functools.partial(
    pl.kernel,
    out_type=jax.ShapeDtypeStruct((_NPAD, _DF), jnp.float32),
    mesh=_mesh,
    scratch_types=[
        pltpu.VMEM_SHARED((_RNG + 8, _DF), jnp.float32),
        pltpu.VMEM((8, 128), jnp.int32),
        pltpu.VMEM((8, 128), jnp.int32),
        pltpu.VMEM((32, 32), jnp.int32),
        pltpu.VMEM((1024,), jnp.float32),
    ] + [pltpu.VMEM((32, _DF), jnp.float32)] * 5
      + [pltpu.SemaphoreType.DMA] * 10,
)
def _propagate(tin, src2, dst2, val2, tout, acc, sidx, draw, didx, cval,
               rows0, rows1, rows2, rows3, rows4,
               g0, g1, g2, g3, g4, s0, s1, s2, s3, s4):
    cid = lax.axis_index("c")
    sid = lax.axis_index("s")
    zero16 = jnp.zeros((16,), jnp.float32)
    rows = (rows0, rows1, rows2, rows3, rows4)
    gsem = (g0, g1, g2, g3, g4)
    ssem = (s0, s1, s2, s3, s4)

    def zfill(i, carry):
        for q in range(8):
            rows0[i, pl.ds(q * 16, 16)] = zero16
            rows1[i, pl.ds(q * 16, 16)] = zero16
        return carry

    for rl in range(2):
        r = 2 * cid + rl
        lo = r * _RNG
        lax.fori_loop(0, 32, zfill, 0)
        for z in range(12):
            pltpu.sync_copy(rows0, acc.at[pl.ds(sid * _WPT + z * 64, 32)])
            pltpu.sync_copy(rows1, acc.at[pl.ds(sid * _WPT + z * 64 + 32, 32)])
        pltpu.sync_copy(rows0.at[pl.ds(0, 16)], acc.at[pl.ds(sid * _WPT + 768, 16)])
        plsc.subcore_barrier()

        def chunk(c, carry):
            base = sid * _RPT + 8 * c
            pltpu.sync_copy(src2.at[pl.ds(base, 8)], sidx)
            pltpu.sync_copy(dst2.at[pl.ds(base, 8)], draw)
            pltpu.sync_copy(val2.at[pl.ds(base * 128, 1024)], cval)

            def locdst(g, ic):
                j = g // 8
                col = (g - j * 8) * 16
                v = draw[j, pl.ds(col, 16)]
                m = (v >= lo) & (v < lo + _RNG)
                didx[g // 2, pl.ds((g - (g // 2) * 2) * 16, 16)] = (
                    jnp.where(m, v - lo, _RNG)
                )
                return ic

            lax.fori_loop(0, 64, locdst, 0)

            def fire_gather(h):
                b = h % 5
                return pltpu.async_copy(
                    tin.at[sidx.at[h // 4, pl.ds((h % 4) * 32, 32)]],
                    rows[b],
                    gsem[b],
                )

            hg = {h: fire_gather(h) for h in range(3)}
            hs = {}
            for h in range(32):
                b = h % 5
                rb = rows[b]
                hg[h].wait()

                def scale_group(g, ic, _h=h, _rb=rb):
                    val16 = cval[pl.ds(_h * 32 + g * 16, 16)]

                    def lane(l, c2):
                        vs = val16[jnp.full((16,), l, jnp.int32)]
                        e = g * 16 + l
                        for q in range(8):
                            fs = pl.ds(q * 16, 16)
                            _rb[e, fs] = _rb[e, fs] * vs
                        return c2

                    return lax.fori_loop(0, 16, lane, ic)

                lax.fori_loop(0, 2, scale_group, 0)
                hs[h] = pltpu.async_copy(rb, acc.at[didx.at[h]], ssem[b], add=True)
                nx = h + 3
                if nx < 32:
                    if h >= 2:
                        hs[h - 2].wait()
                    hg[nx] = fire_gather(nx)
            for h in range(27, 32):
                hs[h].wait()
            return carry

        lax.fori_loop(0, _CHUNKS, chunk, 0)
        plsc.subcore_barrier()
        pltpu.sync_copy(
            acc.at[pl.ds(sid * _WPT, _WPT)],
            tout.at[pl.ds(lo + sid * _WPT, _WPT)],
        )
        plsc.subcore_barrier()


@functools.partial(
    pl.kernel,
    out_type=jax.ShapeDtypeStruct((_NC * _BATCH, _DF), jnp.float32),
    mesh=_mesh,
    scratch_types=[
        pltpu.VMEM((64,), jnp.int32),
        pltpu.VMEM((64, _DF), jnp.float32),
        pltpu.VMEM((64, _DF), jnp.float32),
        pltpu.SemaphoreType.DMA,
    ],
)
def _user_gather(t0, t1, t2, t3, uidx, uout, uidxv, rowbuf, ubuf, usem):
    cid = lax.axis_index("c")
    sid = lax.axis_index("s")
    ub = sid * 64

    pltpu.sync_copy(uidx.at[pl.ds(ub, 64)], uidxv)

    def grab(tab, accumulate):
        pltpu.async_copy(tab.at[uidxv], rowbuf, usem).wait()

        def accrow(rr, carry):
            for q in range(8):
                fs = pl.ds(q * 16, 16)
                if accumulate:
                    ubuf[rr, fs] = ubuf[rr, fs] + rowbuf[rr, fs]
                else:
                    ubuf[rr, fs] = rowbuf[rr, fs]
            return carry

        lax.fori_loop(0, 64, accrow, 0)

    @pl.when(cid == 0)
    def _():
        grab(t0, False)
        grab(t1, True)

    @pl.when(cid == 1)
    def _():
        grab(t2, False)
        grab(t3, True)

    def fin(rr, carry):
        for q in range(8):
            fs = pl.ds(q * 16, 16)
            ubuf[rr, fs] = ubuf[rr, fs] * 0.25
        return carry

    lax.fori_loop(0, 64, fin, 0)
    pltpu.sync_copy(ubuf, uout.at[pl.ds(cid * _BATCH + ub, 64)])


def _items_mean_body(i0_ref, i1_ref, i2_ref, i3_ref, out_ref):
    out_ref[...] = (i0_ref[...] + i1_ref[...] + i2_ref[...] + i3_ref[...]) * 0.25


def _items_mean(t0, t1, t2, t3):
    spec = pl.BlockSpec((1000, _DF), lambda i: (25 + i, 0))
    return pl.pallas_call(
        _items_mean_body,
        grid=(_N_ITEMS // 1000,),
        in_specs=[spec, spec, spec, spec],
        out_specs=pl.BlockSpec((1000, _DF), lambda i: (i, 0)),
        out_shape=jax.ShapeDtypeStruct((_N_ITEMS, _DF), jnp.float32),
    )(t0, t1, t2, t3)


def _rating_body(u_ref, items_ref, rat_ref, orig_ref):
    u = u_ref[0] + u_ref[1]
    ua = u[:, :_D]
    ub = u[:, _D:]
    items_a = items_ref[:, :_D]
    items_b = items_ref[:, _D:]
    dn = (((1,), (1,)), ((), ()))
    a = jax.nn.sigmoid(lax.dot_general(ua, items_a, dn, preferred_element_type=jnp.float32))
    b = jax.nn.sigmoid(lax.dot_general(ub, items_b, dn, preferred_element_type=jnp.float32))
    rat_ref[...] = a + b
    orig_ref[...] = b


_BU = 64


def _rating(u, items_mean):
    return pl.pallas_call(
        _rating_body,
        grid=(_BATCH // _BU,),
        in_specs=[
            pl.BlockSpec((_NC, _BU, _DF), lambda i: (0, i, 0)),
            pl.BlockSpec((_N_ITEMS, _DF), lambda i: (0, 0)),
        ],
        out_specs=[
            pl.BlockSpec((_BU, _N_ITEMS), lambda i: (i, 0)),
            pl.BlockSpec((_BU, _N_ITEMS), lambda i: (i, 0)),
        ],
        out_shape=[
            jax.ShapeDtypeStruct((_BATCH, _N_ITEMS), jnp.float32),
            jax.ShapeDtypeStruct((_BATCH, _N_ITEMS), jnp.float32),
        ],
    )(u, items_mean)


def kernel(users, edge_index, edge_vals, emb_user_conf, emb_user_nonconf,
           emb_item_pop, emb_item_nonpop):
    src = edge_index[0]
    dst = edge_index[1]
    tower_a = jnp.concatenate([emb_user_conf, emb_item_pop], axis=0)
    tower_b = jnp.concatenate([emb_user_nonconf, emb_item_nonpop], axis=0)
    comb = jnp.concatenate([tower_a, tower_b], axis=1)
    t0 = jnp.pad(comb, ((0, _NPAD - _N), (0, 0)))

    pad = _EPAD - _E
    src2 = jnp.pad(src, (0, pad)).reshape(_ER, 128)
    dst2 = jnp.pad(dst, (0, pad)).reshape(_ER, 128)
    val2 = jnp.pad(edge_vals, (0, pad))

    tables = [t0]
    for _ in range(_NLAYERS):
        tables.append(_propagate(tables[-1], src2, dst2, val2))

    u = _user_gather(tables[0], tables[1], tables[2], tables[3], users)
    u = u.reshape(_NC, _BATCH, _DF)

    im = _items_mean(tables[0], tables[1], tables[2], tables[3])
    rating, rating_orig = _rating(u, im)
    return rating, rating_orig


# speedup vs baseline: 1.3390x; 1.1730x over previous
import functools

import jax
import jax.numpy as jnp
from jax import lax
from jax.experimental import pallas as pl
from jax.experimental.pallas import tpu as pltpu
from jax.experimental.pallas import tpu_sc as plsc

_N_USERS = 25000
_N_ITEMS = 25000
_N = _N_USERS + _N_ITEMS
_E = 800000
_D = 64
_DF = 128
_NLAYERS = 3
_BATCH = 1024

_NC = 2
_NS = 16
_ECAP = 212992
_ERB = _ECAP // 128
_RPT = _ERB // _NS
_CHUNKS = _RPT // 8
_NPAD = 50176
_RNG = _NPAD // 4
_WPT = _RNG // _NS

_mesh = plsc.VectorSubcoreMesh(core_axis_name="c", subcore_axis_name="s")


@functools.partial(
    pl.kernel,
    out_type=jax.ShapeDtypeStruct((_NPAD, _DF), jnp.float32),
    mesh=_mesh,
    scratch_types=[
        pltpu.VMEM_SHARED((_RNG + 8, _DF), jnp.float32),
        pltpu.VMEM((8, 128), jnp.int32),
        pltpu.VMEM((8, 128), jnp.int32),
        pltpu.VMEM((32, 32), jnp.int32),
        pltpu.VMEM((1024,), jnp.float32),
    ] + [pltpu.VMEM((32, _DF), jnp.float32)] * 5
      + [pltpu.SemaphoreType.DMA] * 10,
)
def _propagate(tin, src2, dst2, val2, tout, acc, sidx, draw, didx, cval,
               rows0, rows1, rows2, rows3, rows4,
               g0, g1, g2, g3, g4, s0, s1, s2, s3, s4):
    cid = lax.axis_index("c")
    sid = lax.axis_index("s")
    zero16 = jnp.zeros((16,), jnp.float32)
    rows = (rows0, rows1, rows2, rows3, rows4)
    gsem = (g0, g1, g2, g3, g4)
    ssem = (s0, s1, s2, s3, s4)

    def zfill(i, carry):
        for q in range(8):
            rows0[i, pl.ds(q * 16, 16)] = zero16
            rows1[i, pl.ds(q * 16, 16)] = zero16
        return carry

    for rl in range(2):
        r = 2 * cid + rl
        lo = r * _RNG
        lax.fori_loop(0, 32, zfill, 0)
        for z in range(12):
            pltpu.sync_copy(rows0, acc.at[pl.ds(sid * _WPT + z * 64, 32)])
            pltpu.sync_copy(rows1, acc.at[pl.ds(sid * _WPT + z * 64 + 32, 32)])
        pltpu.sync_copy(rows0.at[pl.ds(0, 16)], acc.at[pl.ds(sid * _WPT + 768, 16)])
        plsc.subcore_barrier()

        def chunk(c, carry):
            base = r * _ERB + sid * _RPT + 8 * c
            pltpu.sync_copy(src2.at[pl.ds(base, 8)], sidx)
            pltpu.sync_copy(dst2.at[pl.ds(base, 8)], draw)
            pltpu.sync_copy(val2.at[pl.ds(base * 128, 1024)], cval)

            def locdst(g, ic):
                j = g // 8
                col = (g - j * 8) * 16
                v = draw[j, pl.ds(col, 16)]
                m = (v >= lo) & (v < lo + _RNG)
                didx[g // 2, pl.ds((g - (g // 2) * 2) * 16, 16)] = (
                    jnp.where(m, v - lo, _RNG)
                )
                return ic

            lax.fori_loop(0, 64, locdst, 0)

            def fire_gather(h):
                b = h % 5
                return pltpu.async_copy(
                    tin.at[sidx.at[h // 4, pl.ds((h % 4) * 32, 32)]],
                    rows[b],
                    gsem[b],
                )

            hg = {h: fire_gather(h) for h in range(3)}
            hs = {}
            for h in range(32):
                b = h % 5
                rb = rows[b]
                hg[h].wait()

                def scale_group(g, ic, _h=h, _rb=rb):
                    val16 = cval[pl.ds(_h * 32 + g * 16, 16)]

                    def lane(l, c2):
                        vs = val16[jnp.full((16,), l, jnp.int32)]
                        e = g * 16 + l
                        for q in range(8):
                            fs = pl.ds(q * 16, 16)
                            _rb[e, fs] = _rb[e, fs] * vs
                        return c2

                    return lax.fori_loop(0, 16, lane, ic)

                lax.fori_loop(0, 2, scale_group, 0)
                hs[h] = pltpu.async_copy(rb, acc.at[didx.at[h]], ssem[b], add=True)
                nx = h + 3
                if nx < 32:
                    if h >= 2:
                        hs[h - 2].wait()
                    hg[nx] = fire_gather(nx)
            for h in range(27, 32):
                hs[h].wait()
            return carry

        lax.fori_loop(0, _CHUNKS, chunk, 0)
        plsc.subcore_barrier()
        pltpu.sync_copy(
            acc.at[pl.ds(sid * _WPT, _WPT)],
            tout.at[pl.ds(lo + sid * _WPT, _WPT)],
        )
        plsc.subcore_barrier()


@functools.partial(
    pl.kernel,
    out_type=jax.ShapeDtypeStruct((_NC * _BATCH, _DF), jnp.float32),
    mesh=_mesh,
    scratch_types=[
        pltpu.VMEM((64,), jnp.int32),
        pltpu.VMEM((64, _DF), jnp.float32),
        pltpu.VMEM((64, _DF), jnp.float32),
        pltpu.SemaphoreType.DMA,
    ],
)
def _user_gather(t0, t1, t2, t3, uidx, uout, uidxv, rowbuf, ubuf, usem):
    cid = lax.axis_index("c")
    sid = lax.axis_index("s")
    ub = sid * 64

    pltpu.sync_copy(uidx.at[pl.ds(ub, 64)], uidxv)

    def grab(tab, accumulate):
        pltpu.async_copy(tab.at[uidxv], rowbuf, usem).wait()

        def accrow(rr, carry):
            for q in range(8):
                fs = pl.ds(q * 16, 16)
                if accumulate:
                    ubuf[rr, fs] = ubuf[rr, fs] + rowbuf[rr, fs]
                else:
                    ubuf[rr, fs] = rowbuf[rr, fs]
            return carry

        lax.fori_loop(0, 64, accrow, 0)

    @pl.when(cid == 0)
    def _():
        grab(t0, False)
        grab(t1, True)

    @pl.when(cid == 1)
    def _():
        grab(t2, False)
        grab(t3, True)

    def fin(rr, carry):
        for q in range(8):
            fs = pl.ds(q * 16, 16)
            ubuf[rr, fs] = ubuf[rr, fs] * 0.25
        return carry

    lax.fori_loop(0, 64, fin, 0)
    pltpu.sync_copy(ubuf, uout.at[pl.ds(cid * _BATCH + ub, 64)])


def _items_mean_body(i0_ref, i1_ref, i2_ref, i3_ref, out_ref):
    out_ref[...] = (i0_ref[...] + i1_ref[...] + i2_ref[...] + i3_ref[...]) * 0.25


def _items_mean(t0, t1, t2, t3):
    spec = pl.BlockSpec((1000, _DF), lambda i: (25 + i, 0))
    return pl.pallas_call(
        _items_mean_body,
        grid=(_N_ITEMS // 1000,),
        in_specs=[spec, spec, spec, spec],
        out_specs=pl.BlockSpec((1000, _DF), lambda i: (i, 0)),
        out_shape=jax.ShapeDtypeStruct((_N_ITEMS, _DF), jnp.float32),
    )(t0, t1, t2, t3)


def _rating_body(u_ref, items_ref, rat_ref, orig_ref):
    u = u_ref[0] + u_ref[1]
    ua = u[:, :_D]
    ub = u[:, _D:]
    items_a = items_ref[:, :_D]
    items_b = items_ref[:, _D:]
    dn = (((1,), (1,)), ((), ()))
    a = jax.nn.sigmoid(lax.dot_general(ua, items_a, dn, preferred_element_type=jnp.float32))
    b = jax.nn.sigmoid(lax.dot_general(ub, items_b, dn, preferred_element_type=jnp.float32))
    rat_ref[...] = a + b
    orig_ref[...] = b


_BU = 64


def _rating(u, items_mean):
    return pl.pallas_call(
        _rating_body,
        grid=(_BATCH // _BU,),
        in_specs=[
            pl.BlockSpec((_NC, _BU, _DF), lambda i: (0, i, 0)),
            pl.BlockSpec((_N_ITEMS, _DF), lambda i: (0, 0)),
        ],
        out_specs=[
            pl.BlockSpec((_BU, _N_ITEMS), lambda i: (i, 0)),
            pl.BlockSpec((_BU, _N_ITEMS), lambda i: (i, 0)),
        ],
        out_shape=[
            jax.ShapeDtypeStruct((_BATCH, _N_ITEMS), jnp.float32),
            jax.ShapeDtypeStruct((_BATCH, _N_ITEMS), jnp.float32),
        ],
    )(u, items_mean)


def kernel(users, edge_index, edge_vals, emb_user_conf, emb_user_nonconf,
           emb_item_pop, emb_item_nonpop):
    src = edge_index[0]
    dst = edge_index[1]
    tower_a = jnp.concatenate([emb_user_conf, emb_item_pop], axis=0)
    tower_b = jnp.concatenate([emb_user_nonconf, emb_item_nonpop], axis=0)
    comb = jnp.concatenate([tower_a, tower_b], axis=1)
    t0 = jnp.pad(comb, ((0, _NPAD - _N), (0, 0)))

    bkt = dst // _RNG
    pos = jnp.zeros((_E,), jnp.int32)
    for rr in range(4):
        is_r = bkt == rr
        rank = jnp.cumsum(is_r.astype(jnp.int32)) - 1
        pos = jnp.where(is_r, rr * _ECAP + rank, pos)
    eid = jnp.full((4 * _ECAP,), _E, jnp.int32).at[pos].set(
        jnp.arange(_E, dtype=jnp.int32), mode="drop")
    src_ext = jnp.concatenate([src, jnp.zeros((1,), jnp.int32)])
    dst_ext = jnp.concatenate([dst, jnp.zeros((1,), jnp.int32)])
    val_ext = jnp.concatenate([edge_vals, jnp.zeros((1,), jnp.float32)])
    src2 = src_ext[eid].reshape(4 * _ERB, 128)
    dst2 = dst_ext[eid].reshape(4 * _ERB, 128)
    val2 = val_ext[eid]

    tables = [t0]
    for _ in range(_NLAYERS):
        tables.append(_propagate(tables[-1], src2, dst2, val2))

    u = _user_gather(tables[0], tables[1], tables[2], tables[3], users)
    u = u.reshape(_NC, _BATCH, _DF)

    im = _items_mean(tables[0], tables[1], tables[2], tables[3])
    rating, rating_orig = _rating(u, im)
    return rating, rating_orig


# speedup vs baseline: 1.7785x; 1.3283x over previous
import functools

import jax
import jax.numpy as jnp
from jax import lax
from jax.experimental import pallas as pl
from jax.experimental.pallas import tpu as pltpu
from jax.experimental.pallas import tpu_sc as plsc

_N_USERS = 25000
_N_ITEMS = 25000
_N = _N_USERS + _N_ITEMS
_E = 800000
_D = 64
_DF = 128
_NLAYERS = 3
_BATCH = 1024

_NC = 2
_NS = 16
_ECAP = 212992
_ERB = _ECAP // 128
_RPT = _ERB // _NS
_CHUNKS = _RPT // 8
_NPAD = 50176
_RNG = _NPAD // 4
_WPT = _RNG // _NS

_mesh = plsc.VectorSubcoreMesh(core_axis_name="c", subcore_axis_name="s")


@functools.partial(
    pl.kernel,
    out_type=jax.ShapeDtypeStruct((_NPAD, _DF), jnp.float32),
    mesh=_mesh,
    scratch_types=[
        pltpu.VMEM_SHARED((_RNG + 8, _DF), jnp.float32),
        pltpu.VMEM((8, 128), jnp.int32),
        pltpu.VMEM((8, 128), jnp.int32),
        pltpu.VMEM((32, 32), jnp.int32),
        pltpu.VMEM((1024,), jnp.float32),
    ] + [pltpu.VMEM((32, _DF), jnp.float32)] * 5
      + [pltpu.SemaphoreType.DMA] * 10,
)
def _propagate(tin, src2, dst2, val2, tout, acc, sidx, draw, didx, cval,
               rows0, rows1, rows2, rows3, rows4,
               g0, g1, g2, g3, g4, s0, s1, s2, s3, s4):
    cid = lax.axis_index("c")
    sid = lax.axis_index("s")
    zero16 = jnp.zeros((16,), jnp.float32)
    rows = (rows0, rows1, rows2, rows3, rows4)
    gsem = (g0, g1, g2, g3, g4)
    ssem = (s0, s1, s2, s3, s4)

    def zfill(i, carry):
        for q in range(8):
            rows0[i, pl.ds(q * 16, 16)] = zero16
            rows1[i, pl.ds(q * 16, 16)] = zero16
        return carry

    for rl in range(2):
        r = 2 * cid + rl
        lo = r * _RNG
        lax.fori_loop(0, 32, zfill, 0)
        for z in range(12):
            pltpu.sync_copy(rows0, acc.at[pl.ds(sid * _WPT + z * 64, 32)])
            pltpu.sync_copy(rows1, acc.at[pl.ds(sid * _WPT + z * 64 + 32, 32)])
        pltpu.sync_copy(rows0.at[pl.ds(0, 16)], acc.at[pl.ds(sid * _WPT + 768, 16)])
        plsc.subcore_barrier()

        def chunk(c, carry):
            base = r * _ERB + sid * _RPT + 8 * c
            pltpu.sync_copy(src2.at[pl.ds(base, 8)], sidx)
            pltpu.sync_copy(dst2.at[pl.ds(base, 8)], draw)
            pltpu.sync_copy(val2.at[pl.ds(base * 128, 1024)], cval)

            def locdst(g, ic):
                j = g // 8
                col = (g - j * 8) * 16
                v = draw[j, pl.ds(col, 16)]
                m = (v >= lo) & (v < lo + _RNG)
                didx[g // 2, pl.ds((g - (g // 2) * 2) * 16, 16)] = (
                    jnp.where(m, v - lo, _RNG)
                )
                return ic

            lax.fori_loop(0, 64, locdst, 0)

            def fire_gather(h):
                b = h % 5
                return pltpu.async_copy(
                    tin.at[sidx.at[h // 4, pl.ds((h % 4) * 32, 32)]],
                    rows[b],
                    gsem[b],
                )

            hg = {h: fire_gather(h) for h in range(3)}
            hs = {}
            for h in range(32):
                b = h % 5
                rb = rows[b]
                hg[h].wait()

                def scale_group(g, ic, _h=h, _rb=rb):
                    val16 = cval[pl.ds(_h * 32 + g * 16, 16)]

                    def lane(l, c2):
                        vs = val16[jnp.full((16,), l, jnp.int32)]
                        e = g * 16 + l
                        for q in range(8):
                            fs = pl.ds(q * 16, 16)
                            _rb[e, fs] = _rb[e, fs] * vs
                        return c2

                    return lax.fori_loop(0, 16, lane, ic)

                lax.fori_loop(0, 2, scale_group, 0)
                hs[h] = pltpu.async_copy(rb, acc.at[didx.at[h]], ssem[b], add=True)
                nx = h + 3
                if nx < 32:
                    if h >= 2:
                        hs[h - 2].wait()
                    hg[nx] = fire_gather(nx)
            for h in range(27, 32):
                hs[h].wait()
            return carry

        lax.fori_loop(0, _CHUNKS, chunk, 0)
        plsc.subcore_barrier()
        pltpu.sync_copy(
            acc.at[pl.ds(sid * _WPT, _WPT)],
            tout.at[pl.ds(lo + sid * _WPT, _WPT)],
        )
        plsc.subcore_barrier()


@functools.partial(
    pl.kernel,
    out_type=jax.ShapeDtypeStruct((_NC * _BATCH, _DF), jnp.float32),
    mesh=_mesh,
    scratch_types=[
        pltpu.VMEM((64,), jnp.int32),
        pltpu.VMEM((64, _DF), jnp.float32),
        pltpu.VMEM((64, _DF), jnp.float32),
        pltpu.SemaphoreType.DMA,
    ],
)
def _user_gather(t0, t1, t2, t3, uidx, uout, uidxv, rowbuf, ubuf, usem):
    cid = lax.axis_index("c")
    sid = lax.axis_index("s")
    ub = sid * 64

    pltpu.sync_copy(uidx.at[pl.ds(ub, 64)], uidxv)

    def grab(tab, accumulate):
        pltpu.async_copy(tab.at[uidxv], rowbuf, usem).wait()

        def accrow(rr, carry):
            for q in range(8):
                fs = pl.ds(q * 16, 16)
                if accumulate:
                    ubuf[rr, fs] = ubuf[rr, fs] + rowbuf[rr, fs]
                else:
                    ubuf[rr, fs] = rowbuf[rr, fs]
            return carry

        lax.fori_loop(0, 64, accrow, 0)

    @pl.when(cid == 0)
    def _():
        grab(t0, False)
        grab(t1, True)

    @pl.when(cid == 1)
    def _():
        grab(t2, False)
        grab(t3, True)

    def fin(rr, carry):
        for q in range(8):
            fs = pl.ds(q * 16, 16)
            ubuf[rr, fs] = ubuf[rr, fs] * 0.25
        return carry

    lax.fori_loop(0, 64, fin, 0)
    pltpu.sync_copy(ubuf, uout.at[pl.ds(cid * _BATCH + ub, 64)])


def _items_mean_body(i0_ref, i1_ref, i2_ref, i3_ref, out_ref):
    out_ref[...] = (i0_ref[...] + i1_ref[...] + i2_ref[...] + i3_ref[...]) * 0.25


def _items_mean(t0, t1, t2, t3):
    spec = pl.BlockSpec((1000, _DF), lambda i: (25 + i, 0))
    return pl.pallas_call(
        _items_mean_body,
        grid=(_N_ITEMS // 1000,),
        in_specs=[spec, spec, spec, spec],
        out_specs=pl.BlockSpec((1000, _DF), lambda i: (i, 0)),
        out_shape=jax.ShapeDtypeStruct((_N_ITEMS, _DF), jnp.float32),
    )(t0, t1, t2, t3)


def _rating_body(u_ref, items_ref, rat_ref, orig_ref):
    u = u_ref[0] + u_ref[1]
    ua = u[:, :_D]
    ub = u[:, _D:]
    items_a = items_ref[:, :_D]
    items_b = items_ref[:, _D:]
    dn = (((1,), (1,)), ((), ()))
    a = jax.nn.sigmoid(lax.dot_general(ua, items_a, dn, preferred_element_type=jnp.float32))
    b = jax.nn.sigmoid(lax.dot_general(ub, items_b, dn, preferred_element_type=jnp.float32))
    rat_ref[...] = a + b
    orig_ref[...] = b


_BU = 64


def _rating(u, items_mean):
    return pl.pallas_call(
        _rating_body,
        grid=(_BATCH // _BU,),
        in_specs=[
            pl.BlockSpec((_NC, _BU, _DF), lambda i: (0, i, 0)),
            pl.BlockSpec((_N_ITEMS, _DF), lambda i: (0, 0)),
        ],
        out_specs=[
            pl.BlockSpec((_BU, _N_ITEMS), lambda i: (i, 0)),
            pl.BlockSpec((_BU, _N_ITEMS), lambda i: (i, 0)),
        ],
        out_shape=[
            jax.ShapeDtypeStruct((_BATCH, _N_ITEMS), jnp.float32),
            jax.ShapeDtypeStruct((_BATCH, _N_ITEMS), jnp.float32),
        ],
    )(u, items_mean)


def kernel(users, edge_index, edge_vals, emb_user_conf, emb_user_nonconf,
           emb_item_pop, emb_item_nonpop):
    src = edge_index[0]
    dst = edge_index[1]
    tower_a = jnp.concatenate([emb_user_conf, emb_item_pop], axis=0)
    tower_b = jnp.concatenate([emb_user_nonconf, emb_item_nonpop], axis=0)
    comb = jnp.concatenate([tower_a, tower_b], axis=1)
    t0 = jnp.pad(comb, ((0, _NPAD - _N), (0, 0)))

    npad = 4 * _ECAP - _E
    bkt = dst // _RNG
    counts = jnp.zeros((4,), jnp.int32).at[bkt].add(1)
    deficit_cum = jnp.cumsum(_ECAP - counts)
    pad_bkt = jnp.searchsorted(
        deficit_cum, jnp.arange(npad, dtype=jnp.int32), side="right"
    ).astype(jnp.int32)
    key = jnp.concatenate([bkt, pad_bkt])
    src_e = jnp.concatenate([src, jnp.zeros((npad,), jnp.int32)])
    dst_e = jnp.concatenate([dst, jnp.zeros((npad,), jnp.int32)])
    val_e = jnp.concatenate([edge_vals, jnp.zeros((npad,), jnp.float32)])
    _, srcb, dstb, valb = lax.sort((key, src_e, dst_e, val_e), num_keys=1)
    src2 = srcb.reshape(4 * _ERB, 128)
    dst2 = dstb.reshape(4 * _ERB, 128)
    val2 = valb

    tables = [t0]
    for _ in range(_NLAYERS):
        tables.append(_propagate(tables[-1], src2, dst2, val2))

    u = _user_gather(tables[0], tables[1], tables[2], tables[3], users)
    u = u.reshape(_NC, _BATCH, _DF)

    im = _items_mean(tables[0], tables[1], tables[2], tables[3])
    rating, rating_orig = _rating(u, im)
    return rating, rating_orig
